# Initial kernel scaffold; baseline (speedup 1.0000x reference)
import jax
import jax.numpy as jnp
from jax.experimental import pallas as pl


def kernel(node_states, edge_types, enc1_W1, enc1_b1, enc1_W2, enc1_b2, enc2_W1, enc2_b1, enc2_W2, enc2_b2, dec_W1, dec_b1, dec_W2, dec_b2):
    raise NotImplementedError("write your pallas kernel here")


# speedup vs baseline: 9.1241x; 9.1241x over previous
import jax
import jax.numpy as jnp
from jax.experimental import pallas as pl

_N = 64
_D = 64


def _body(x_ref, wg_ref,
          e1w1_ref, e1b1_ref, e1w2_ref, e1b2_ref,
          e2w1_ref, e2b1_ref, e2w2_ref, e2b2_ref,
          dw1_ref, db1_ref, dw2_ref, db2_ref,
          out_ref):
    x = x_ref[0]
    wg = wg_ref[0]

    def encode(w1_ref, b1_ref, w2_ref, b2_ref):
        w1 = w1_ref[...]
        a = jnp.dot(x, w1[:_D], preferred_element_type=jnp.float32)
        a = a + b1_ref[...]
        b = jnp.dot(x, w1[_D:], preferred_element_type=jnp.float32)
        h = jax.nn.relu(a[:, None, :] + b[None, :, :])
        h = h.reshape(_N * _N, 96)
        m = jnp.dot(h, w2_ref[...], preferred_element_type=jnp.float32)
        m = jax.nn.relu(m + b2_ref[...])
        return m.reshape(_N, _N, _D)

    m1 = encode(e1w1_ref, e1b1_ref, e1w2_ref, e1b2_ref)
    m2 = encode(e2w1_ref, e2b1_ref, e2w2_ref, e2b2_ref)

    weighted = wg[0][:, :, None] * m1 + wg[1][:, :, None] * m2
    node_msg = jnp.sum(weighted, axis=0)

    dw1 = dw1_ref[...]
    d1 = (jnp.dot(x, dw1[:_D], preferred_element_type=jnp.float32)
          + jnp.dot(node_msg, dw1[_D:], preferred_element_type=jnp.float32)
          + db1_ref[...])
    d1 = jax.nn.relu(d1)
    out = jnp.dot(d1, dw2_ref[...], preferred_element_type=jnp.float32)
    out_ref[0] = jax.nn.relu(out + db2_ref[...])


def kernel(node_states, edge_types, enc1_W1, enc1_b1, enc1_W2, enc1_b2,
           enc2_W1, enc2_b1, enc2_W2, enc2_b2, dec_W1, dec_b1, dec_W2, dec_b2):
    B = node_states.shape[0]
    x = node_states.reshape(B, _N, _D)

    et = edge_types[:, :, 1:, 0]
    et = jnp.transpose(et, (0, 2, 1))
    et = et.reshape(B, 2, _N - 1, _N)
    et = jnp.pad(et, ((0, 0), (0, 0), (0, 0), (0, 1)))
    et = et.reshape(B, 2, (_N - 1) * (_N + 1))
    et = jnp.pad(et, ((0, 0), (0, 0), (1, 0)))
    wgrid = et.reshape(B, 2, _N, _N)

    def full(a):
        return pl.BlockSpec(a.shape, lambda i: (0,) * a.ndim)

    e1b1 = enc1_b1.reshape(1, -1)
    e1b2 = enc1_b2.reshape(1, -1)
    e2b1 = enc2_b1.reshape(1, -1)
    e2b2 = enc2_b2.reshape(1, -1)
    db1 = dec_b1.reshape(1, -1)
    db2 = dec_b2.reshape(1, -1)

    weights = (enc1_W1, e1b1, enc1_W2, e1b2,
               enc2_W1, e2b1, enc2_W2, e2b2,
               dec_W1, db1, dec_W2, db2)

    out = pl.pallas_call(
        _body,
        grid=(B,),
        in_specs=[
            pl.BlockSpec((1, _N, _D), lambda i: (i, 0, 0)),
            pl.BlockSpec((1, 2, _N, _N), lambda i: (i, 0, 0, 0)),
        ] + [full(w) for w in weights],
        out_specs=pl.BlockSpec((1, _N, _D), lambda i: (i, 0, 0)),
        out_shape=jax.ShapeDtypeStruct((B, _N, _D), jnp.float32),
    )(x, wgrid, *weights)

    return out.reshape(B, _N, 1, _D)


# speedup vs baseline: 10.7429x; 1.1774x over previous
import jax
import jax.numpy as jnp
import numpy as np
from jax.experimental import pallas as pl

_N = 64
_D = 64
_F = 96
_ST = _N * _N
_BP = 4


def _body(xT_ref, wgf_ref, ee_ref,
          w1top_ref, w1bot_ref, b1cat_ref,
          w2T1_ref, b2col1_ref, w2T2_ref, b2col2_ref,
          dw1top_ref, dw1bot_ref, db1_ref, dw2_ref, db2_ref,
          out_ref):
    ee = ee_ref[...]
    for j in range(_BP):
        xT = xT_ref[j]

        a = jnp.dot(w1top_ref[...], xT, preferred_element_type=jnp.float32)
        a = a + b1cat_ref[...]
        b = jnp.dot(w1bot_ref[...], xT, preferred_element_type=jnp.float32)
        ab = jnp.concatenate([a, b], axis=1)
        h = jax.nn.relu(jnp.dot(ab, ee, preferred_element_type=jnp.float32))

        m1 = jnp.dot(w2T1_ref[...], h[:_F], preferred_element_type=jnp.float32)
        m1 = jax.nn.relu(m1 + b2col1_ref[...])
        m2 = jnp.dot(w2T2_ref[...], h[_F:], preferred_element_type=jnp.float32)
        m2 = jax.nn.relu(m2 + b2col2_ref[...])

        wgf = wgf_ref[j]
        wsum = m1 * wgf[0:1, :] + m2 * wgf[1:2, :]

        acc = wsum[:, 0:128]
        for k in range(1, 32):
            acc = acc + wsum[:, k * 128:(k + 1) * 128]
        node_msgT = acc[:, :_N] + acc[:, _N:]

        d1 = jax.lax.dot_general(xT, dw1top_ref[...],
                                 (((0,), (0,)), ((), ())),
                                 preferred_element_type=jnp.float32)
        d1 = d1 + jax.lax.dot_general(node_msgT, dw1bot_ref[...],
                                      (((0,), (0,)), ((), ())),
                                      preferred_element_type=jnp.float32)
        d1 = jax.nn.relu(d1 + db1_ref[...])
        out = jnp.dot(d1, dw2_ref[...], preferred_element_type=jnp.float32)
        out_ref[j] = jax.nn.relu(out + db2_ref[...])


def kernel(node_states, edge_types, enc1_W1, enc1_b1, enc1_W2, enc1_b2,
           enc2_W1, enc2_b1, enc2_W2, enc2_b2, dec_W1, dec_b1, dec_W2, dec_b2):
    B = node_states.shape[0]
    xT = jnp.swapaxes(node_states.reshape(B, _N, _D), 1, 2)

    et = edge_types[:, :, 1:, 0]
    et = jnp.transpose(et, (0, 2, 1))
    et = et.reshape(B, 2, _N - 1, _N)
    et = jnp.pad(et, ((0, 0), (0, 0), (0, 0), (0, 1)))
    et = et.reshape(B, 2, (_N - 1) * (_N + 1))
    wgf = jnp.pad(et, ((0, 0), (0, 0), (1, 0)))

    eye = np.eye(_N, dtype=np.float32)
    ee = jnp.asarray(np.concatenate([np.repeat(eye, _N, axis=1),
                                     np.tile(eye, (1, _N))], axis=0))

    w1top = jnp.concatenate([enc1_W1[:_D].T, enc2_W1[:_D].T], axis=0)
    w1bot = jnp.concatenate([enc1_W1[_D:].T, enc2_W1[_D:].T], axis=0)
    b1cat = jnp.concatenate([enc1_b1, enc2_b1]).reshape(2 * _F, 1)
    w2T1 = enc1_W2.T
    w2T2 = enc2_W2.T
    b2col1 = enc1_b2.reshape(_D, 1)
    b2col2 = enc2_b2.reshape(_D, 1)
    dw1top = dec_W1[:_D]
    dw1bot = dec_W1[_D:]
    db1 = dec_b1.reshape(1, -1)
    db2 = dec_b2.reshape(1, -1)

    def full(a):
        return pl.BlockSpec(a.shape, lambda i: (0,) * a.ndim)

    consts = (ee, w1top, w1bot, b1cat, w2T1, b2col1, w2T2, b2col2,
              dw1top, dw1bot, db1, dec_W2, db2)

    out = pl.pallas_call(
        _body,
        grid=(B // _BP,),
        in_specs=[
            pl.BlockSpec((_BP, _D, _N), lambda i: (i, 0, 0)),
            pl.BlockSpec((_BP, 2, _ST), lambda i: (i, 0, 0)),
        ] + [full(w) for w in consts],
        out_specs=pl.BlockSpec((_BP, _N, _D), lambda i: (i, 0, 0)),
        out_shape=jax.ShapeDtypeStruct((B, _N, _D), jnp.float32),
    )(xT, wgf, *consts)

    return out.reshape(B, _N, 1, _D)
